# baseline (speedup 1.0000x reference)
import functools

import jax
import jax.numpy as jnp
from jax import lax
from jax.experimental import pallas as pl
from jax.experimental.pallas import tpu as pltpu
from jax.experimental.pallas import tpu_sc as plsc

BATCH = 16384
N_FIELDS = 26
VOCAB = 100000
DIM = 50

_INFO = plsc.get_sparse_core_info()
_NC = _INFO.num_cores
_NS = _INFO.num_subcores
_NW = _NC * _NS
_LANES = _INFO.num_lanes

TOTAL_ROWS = BATCH * N_FIELDS
ROWS_PER_W = TOTAL_ROWS // _NW
CHUNK = 832
N_CHUNKS = ROWS_PER_W // CHUNK


def _embed_kernel(labels_hbm, tables_hbm, out_hbm,
                  idx_v, buf0, buf1, sem0, sem1):
    wid = lax.axis_index("s") * _NC + lax.axis_index("c")
    base = wid * ROWS_PER_W

    pltpu.sync_copy(labels_hbm.at[pl.ds(base, ROWS_PER_W)], idx_v)

    lane = lax.iota(jnp.int32, 16)

    def fix(i, carry):
        sl = pl.ds(i * _LANES, _LANES)
        r = base + i * _LANES + lane
        f = lax.rem(r, N_FIELDS)
        idx_v[sl] = idx_v[sl] + f * VOCAB
        return carry

    lax.fori_loop(0, ROWS_PER_W // _LANES, fix, 0)

    bufs = (buf0, buf1)
    sems = (sem0, sem1)

    def gather(c):
        return pltpu.async_copy(
            tables_hbm.at[idx_v.at[pl.ds(c * CHUNK, CHUNK)]],
            bufs[c % 2], sems[c % 2])

    pending = gather(0)
    for c in range(N_CHUNKS):
        nxt = gather(c + 1) if c + 1 < N_CHUNKS else None
        pending.wait()
        pltpu.sync_copy(bufs[c % 2],
                        out_hbm.at[pl.ds(base + c * CHUNK, CHUNK)])
        pending = nxt


@jax.jit
def kernel(labels, tables):
    labels_flat = labels.reshape(TOTAL_ROWS)
    tables_flat = tables.reshape(N_FIELDS * VOCAB, DIM)

    mesh = plsc.VectorSubcoreMesh(core_axis_name="c", subcore_axis_name="s")
    run = pl.kernel(
        _embed_kernel,
        mesh=mesh,
        out_type=jax.ShapeDtypeStruct((TOTAL_ROWS, DIM), jnp.float32),
        scratch_types=[
            pltpu.VMEM((ROWS_PER_W,), jnp.int32),
            pltpu.VMEM((CHUNK, DIM), jnp.float32),
            pltpu.VMEM((CHUNK, DIM), jnp.float32),
            pltpu.SemaphoreType.DMA,
            pltpu.SemaphoreType.DMA,
        ],
        compiler_params=pltpu.CompilerParams(use_tc_tiling_on_sc=False),
    )
    out = run(labels_flat, tables_flat)
    return out.reshape(BATCH, N_FIELDS * DIM)


# speedup vs baseline: 1.0282x; 1.0282x over previous
import jax
import jax.numpy as jnp
from jax import lax
from jax.experimental import pallas as pl
from jax.experimental.pallas import tpu as pltpu
from jax.experimental.pallas import tpu_sc as plsc

BATCH = 16384
N_FIELDS = 26
VOCAB = 100000
DIM = 50

_NC = 2
_NS = 16
_NW = _NC * _NS
_L = 16

TOTAL_ROWS = BATCH * N_FIELDS
ROWS_PER_W = TOTAL_ROWS // _NW
WORDS_PER_W = ROWS_PER_W * DIM
BLOCK_WORDS = 400
BLOCK_ROWS = 8
VECS_PER_BLOCK = BLOCK_WORDS // _L
CHUNK_WORDS = 20800
CHUNK_BLOCKS = CHUNK_WORDS // BLOCK_WORDS
CHUNK_ROWS = CHUNK_WORDS // DIM
N_CHUNKS = WORDS_PER_W // CHUNK_WORDS


def _embed_kernel(labels_hbm, table_hbm, out_hbm,
                  gidx_v, idx0, idx1, val0, val1, sem0, sem1):
    wid = lax.axis_index("s") * _NC + lax.axis_index("c")
    row_base = wid * ROWS_PER_W
    word_base = wid * WORDS_PER_W

    pltpu.sync_copy(labels_hbm.at[pl.ds(row_base, ROWS_PER_W)], gidx_v)
    lane = lax.iota(jnp.int32, _L)

    def fix(i, carry):
        sl = pl.ds(i * _L, _L)
        r = row_base + i * _L + lane
        f = lax.rem(r, N_FIELDS)
        gidx_v[sl] = gidx_v[sl] * DIM + f * (VOCAB * DIM)
        return carry

    lax.fori_loop(0, ROWS_PER_W // _L, fix, 0)

    rowpats = []
    colpats = []
    for v in range(VECS_PER_BLOCK):
        w0 = v * _L
        rb = w0 // DIM
        cross = DIM * (rb + 1) - w0
        if cross < _L:
            rowpat = jnp.where(lane >= cross, rb + 1, rb).astype(jnp.int32)
        else:
            rowpat = jnp.full((_L,), rb, jnp.int32)
        colpat = (w0 + lane) - rowpat * DIM
        rowpats.append(rowpat)
        colpats.append(colpat)

    idx_bufs = (idx0, idx1)
    val_bufs = (val0, val1)
    sems = (sem0, sem1)

    def build_idx(c, idx_ref):
        crow = c * CHUNK_ROWS

        def blk(b, carry):
            rbase = crow + b * BLOCK_ROWS
            for v in range(VECS_PER_BLOCK):
                g = plsc.load_gather(gidx_v, [rbase + rowpats[v]])
                idx_ref[pl.ds(b * BLOCK_WORDS + v * _L, _L)] = g + colpats[v]
            return carry

        lax.fori_loop(0, CHUNK_BLOCKS, blk, 0)

    def start_gather(k, c):
        pltpu.async_copy(table_hbm.at[idx_bufs[k]], val_bufs[k], sems[k])

    def drain(k, c):
        pltpu.make_async_copy(
            table_hbm.at[idx_bufs[k]], val_bufs[k], sems[k]).wait()
        pltpu.sync_copy(val_bufs[k],
                        out_hbm.at[pl.ds(word_base + c * CHUNK_WORDS,
                                         CHUNK_WORDS)])

    def pair(c2, carry):
        for k in (0, 1):
            c = 2 * c2 + k
            build_idx(c, idx_bufs[k])
            start_gather(k, c)
            if k == 0:
                @pl.when(c2 > 0)
                def _():
                    drain(1, 2 * c2 - 1)
            else:
                drain(0, 2 * c2)
        return carry

    lax.fori_loop(0, N_CHUNKS // 2, pair, 0)
    drain(1, N_CHUNKS - 1)


@jax.jit
def kernel(labels, tables):
    labels_flat = labels.reshape(TOTAL_ROWS)
    table_1d = tables.reshape(N_FIELDS * VOCAB * DIM)

    mesh = plsc.VectorSubcoreMesh(core_axis_name="c", subcore_axis_name="s")
    run = pl.kernel(
        _embed_kernel,
        mesh=mesh,
        out_type=jax.ShapeDtypeStruct((TOTAL_ROWS * DIM,), jnp.float32),
        scratch_types=[
            pltpu.VMEM((ROWS_PER_W,), jnp.int32),
            pltpu.VMEM((CHUNK_WORDS,), jnp.int32),
            pltpu.VMEM((CHUNK_WORDS,), jnp.int32),
            pltpu.VMEM((CHUNK_WORDS,), jnp.float32),
            pltpu.VMEM((CHUNK_WORDS,), jnp.float32),
            pltpu.SemaphoreType.DMA,
            pltpu.SemaphoreType.DMA,
        ],
        compiler_params=pltpu.CompilerParams(
            use_tc_tiling_on_sc=False, needs_layout_passes=False),
    )
    out = run(labels_flat, table_1d)
    return out.reshape(BATCH, N_FIELDS * DIM)
